# baseline (speedup 1.0000x reference)
import functools

import jax
import jax.numpy as jnp
from jax import lax
from jax.experimental import pallas as pl
from jax.experimental.pallas import tpu as pltpu
from jax.experimental.pallas import tpu_sc as plsc

_N = 2048
_H = 8
_K = 8192
_DH = 128

_W = 512
_G = _K // _W

_NC = 2
_NS = 16
_NW = _NC * _NS
_ROWS = _N * _H
_RPW = _ROWS // _NW
_GCH = 128
_NCH = _RPW // _GCH


def _argmin_body(x_ref, cb_ref, cbsq_ref, xsq_ref, out_ref):
    h = pl.program_id(0)
    m = jnp.full((_N, _W), jnp.inf, jnp.float32)
    e = jnp.zeros((_N, _W), jnp.int32)
    lane = lax.broadcasted_iota(jnp.int32, (1, _W), 1)
    for g in range(_G):
        dots2 = lax.dot_general(
            x_ref[...], cb_ref[0, g * _W:(g + 1) * _W, :],
            (((1,), (1,)), ((), ())),
            preferred_element_type=jnp.float32,
        )
        d = (xsq_ref[0] + dots2) + cbsq_ref[0, :, g * _W:(g + 1) * _W]
        cond = d < m
        e = jnp.where(cond, lane + g * _W, e)
        m = jnp.minimum(m, d)
    minval = jnp.min(m, axis=1, keepdims=True)
    idx = jnp.min(jnp.where(m == minval, e, _K), axis=1, keepdims=True)
    out_ref[0] = idx + h * _K


def _tc_argmin(x2dm, codebook, cb_sq, x_sq, interpret=False):
    return pl.pallas_call(
        _argmin_body,
        grid=(_H,),
        in_specs=[
            pl.BlockSpec((_N, _DH), lambda h: (0, h)),
            pl.BlockSpec((1, _K, _DH), lambda h: (h, 0, 0)),
            pl.BlockSpec((1, 1, _K), lambda h: (h, 0, 0)),
            pl.BlockSpec((1, _N, 1), lambda h: (h, 0, 0)),
        ],
        out_specs=pl.BlockSpec((1, _N, 1), lambda h: (h, 0, 0)),
        out_shape=jax.ShapeDtypeStruct((_H, _N, 1), jnp.int32),
        compiler_params=pltpu.CompilerParams(
            dimension_semantics=("parallel",)),
        interpret=interpret,
    )(x2dm, codebook, cb_sq, x_sq)


def _sc_gather_body(table_hbm, idx_hbm, out_hbm, list_v, rows_v, sem):
    wid = lax.axis_index("s") * _NC + lax.axis_index("c")
    base = wid * _RPW
    pltpu.sync_copy(idx_hbm.at[pl.ds(wid * _NCH, _NCH)], list_v)
    for c in range(_NCH):
        pltpu.async_copy(table_hbm.at[list_v.at[c]], rows_v, sem).wait()
        pltpu.sync_copy(rows_v, out_hbm.at[pl.ds(base + c * _GCH, _GCH)])


@functools.lru_cache(maxsize=1)
def _sc_gather():
    return pl.kernel(
        _sc_gather_body,
        out_type=jax.ShapeDtypeStruct((_ROWS, _DH), jnp.float32),
        mesh=plsc.VectorSubcoreMesh(
            core_axis_name="c", subcore_axis_name="s",
            num_cores=_NC, num_subcores=_NS),
        scratch_types=[
            pltpu.VMEM((_NCH, _GCH), jnp.int32),
            pltpu.VMEM((_GCH, _DH), jnp.float32),
            pltpu.SemaphoreType.DMA,
        ],
    )


def kernel(x, mask, token_type_ids, key_optim, codebook, values):
    b, n, dim = x.shape
    h, k, dh = codebook.shape
    x2d = x.reshape(b * n, dim)
    xh = x.reshape(b * n, h, dh)
    x_sq = jnp.sum(xh * xh, axis=-1)
    cb_sq = jnp.sum(codebook * codebook, axis=-1)

    idx = _tc_argmin(
        x2d * -2.0, codebook,
        cb_sq.reshape(h, 1, k),
        x_sq.T.reshape(h, b * n, 1),
    )

    idx2d = idx.reshape(h, b * n).T.reshape(_ROWS // _GCH, _GCH)
    rows = _sc_gather()(values.reshape(h * k, dh), idx2d)
    return rows.reshape(b, n, h * dh)


# speedup vs baseline: 1.0292x; 1.0292x over previous
import functools

import jax
import jax.numpy as jnp
from jax import lax
from jax.experimental import pallas as pl
from jax.experimental.pallas import tpu as pltpu
from jax.experimental.pallas import tpu_sc as plsc

_N = 2048
_H = 8
_K = 8192
_DH = 128

_W = 512
_G = _K // _W

_NC = 2
_NS = 16
_NW = _NC * _NS
_ROWS = _N * _H
_RPW = _ROWS // _NW
_GCH = 128
_NCH = _RPW // _GCH


def _argmin_body(x_ref, cb_ref, cbsq_ref, xsq_ref, out_ref):
    h = pl.program_id(0)
    m = jnp.full((_N, _W), jnp.inf, jnp.float32)
    e = jnp.zeros((_N, _W), jnp.int32)
    lane = lax.broadcasted_iota(jnp.int32, (1, _W), 1)
    for g in range(_G):
        dots2 = lax.dot_general(
            x_ref[...], cb_ref[0, g * _W:(g + 1) * _W, :],
            (((1,), (1,)), ((), ())),
            preferred_element_type=jnp.float32,
        )
        d = (xsq_ref[0] + dots2) + cbsq_ref[0, :, g * _W:(g + 1) * _W]
        cond = d < m
        e = jnp.where(cond, lane + g * _W, e)
        m = jnp.minimum(m, d)
    minval = jnp.min(m, axis=1, keepdims=True)
    idx = jnp.min(jnp.where(m == minval, e, _K), axis=1, keepdims=True)
    hcol = lax.broadcasted_iota(jnp.int32, (1, _H), 1)
    out_ref[...] = jnp.where(hcol == h, idx + h * _K, out_ref[...])


def _tc_argmin(x2dm, codebook, cb_sq, x_sq, interpret=False):
    return pl.pallas_call(
        _argmin_body,
        grid=(_H,),
        in_specs=[
            pl.BlockSpec((_N, _DH), lambda h: (0, h)),
            pl.BlockSpec((1, _K, _DH), lambda h: (h, 0, 0)),
            pl.BlockSpec((1, 1, _K), lambda h: (h, 0, 0)),
            pl.BlockSpec((1, _N, 1), lambda h: (h, 0, 0)),
        ],
        out_specs=pl.BlockSpec((_N, _H), lambda h: (0, 0)),
        out_shape=jax.ShapeDtypeStruct((_N, _H), jnp.int32),
        interpret=interpret,
    )(x2dm, codebook, cb_sq, x_sq)


def _sc_gather_body(table_hbm, idx_hbm, out_hbm, list_v, rows_v, sem):
    wid = lax.axis_index("s") * _NC + lax.axis_index("c")
    base = wid * _RPW
    pltpu.sync_copy(idx_hbm.at[pl.ds(wid * _NCH, _NCH)], list_v)
    for c in range(_NCH):
        pltpu.async_copy(table_hbm.at[list_v.at[c]], rows_v, sem).wait()
        pltpu.sync_copy(rows_v, out_hbm.at[pl.ds(base + c * _GCH, _GCH)])


@functools.lru_cache(maxsize=1)
def _sc_gather():
    return pl.kernel(
        _sc_gather_body,
        out_type=jax.ShapeDtypeStruct((_ROWS, _DH), jnp.float32),
        mesh=plsc.VectorSubcoreMesh(
            core_axis_name="c", subcore_axis_name="s",
            num_cores=_NC, num_subcores=_NS),
        scratch_types=[
            pltpu.VMEM((_NCH, _GCH), jnp.int32),
            pltpu.VMEM((_GCH, _DH), jnp.float32),
            pltpu.SemaphoreType.DMA,
        ],
    )


def kernel(x, mask, token_type_ids, key_optim, codebook, values):
    b, n, dim = x.shape
    h, k, dh = codebook.shape
    x2d = x.reshape(b * n, dim)
    xh = x.reshape(b * n, h, dh)
    x_sq = jnp.sum(xh * xh, axis=-1)
    cb_sq = jnp.sum(codebook * codebook, axis=-1)

    idx = _tc_argmin(
        x2d * -2.0, codebook,
        cb_sq.reshape(h, 1, k),
        x_sq.T.reshape(h, b * n, 1),
    )

    idx2d = idx.reshape(_ROWS // _GCH, _GCH)
    rows = _sc_gather()(values.reshape(h * k, dh), idx2d)
    return rows.reshape(b, n, h * dh)


# speedup vs baseline: 1.0579x; 1.0280x over previous
import functools

import jax
import jax.numpy as jnp
from jax import lax
from jax.experimental import pallas as pl
from jax.experimental.pallas import tpu as pltpu
from jax.experimental.pallas import tpu_sc as plsc

_N = 2048
_H = 8
_K = 8192
_DH = 128

_W = 512
_G = _K // _W

_NC = 2
_NS = 16
_NW = _NC * _NS
_ROWS = _N * _H
_RPW = _ROWS // _NW
_GCH = 128
_NCH = _RPW // _GCH


def _argmin_body(x_ref, cb_ref, cbsq_ref, xsq_ref, out_ref):
    h = pl.program_id(0)
    m = jnp.full((_N, _W), jnp.inf, jnp.float32)
    e = jnp.zeros((_N, _W), jnp.int32)
    lane = lax.broadcasted_iota(jnp.int32, (1, _W), 1)
    xm = x_ref[...] * -2.0
    for g in range(_G):
        dots2 = lax.dot_general(
            xm, cb_ref[0, g * _W:(g + 1) * _W, :],
            (((1,), (1,)), ((), ())),
            preferred_element_type=jnp.float32,
        )
        d = (xsq_ref[0] + dots2) + cbsq_ref[0, :, g * _W:(g + 1) * _W]
        cond = d < m
        e = jnp.where(cond, lane + g * _W, e)
        m = jnp.minimum(m, d)
    minval = jnp.min(m, axis=1, keepdims=True)
    idx = jnp.min(jnp.where(m == minval, e, _K), axis=1, keepdims=True)
    hcol = lax.broadcasted_iota(jnp.int32, (1, _H), 1)
    out_ref[...] = jnp.where(hcol == h, idx + h * _K, out_ref[...])


def _tc_argmin(x2dm, codebook, cb_sq, x_sq, interpret=False):
    return pl.pallas_call(
        _argmin_body,
        grid=(_H,),
        in_specs=[
            pl.BlockSpec((_N, _DH), lambda h: (0, h)),
            pl.BlockSpec((1, _K, _DH), lambda h: (h, 0, 0)),
            pl.BlockSpec((1, 1, _K), lambda h: (h, 0, 0)),
            pl.BlockSpec((1, _N, 1), lambda h: (h, 0, 0)),
        ],
        out_specs=pl.BlockSpec((_N, _H), lambda h: (0, 0)),
        out_shape=jax.ShapeDtypeStruct((_N, _H), jnp.int32),
        interpret=interpret,
    )(x2dm, codebook, cb_sq, x_sq)


def _sc_gather_body(table_hbm, idx_hbm, out_hbm, list_v, rows_v, sem):
    wid = lax.axis_index("s") * _NC + lax.axis_index("c")
    base = wid * _RPW
    pltpu.sync_copy(idx_hbm.at[pl.ds(wid * _NCH, _NCH)], list_v)
    for c in range(_NCH):
        pltpu.async_copy(table_hbm.at[list_v.at[c]], rows_v, sem).wait()
        pltpu.sync_copy(rows_v, out_hbm.at[pl.ds(base + c * _GCH, _GCH)])


@functools.lru_cache(maxsize=1)
def _sc_gather():
    return pl.kernel(
        _sc_gather_body,
        out_type=jax.ShapeDtypeStruct((_ROWS, _DH), jnp.float32),
        mesh=plsc.VectorSubcoreMesh(
            core_axis_name="c", subcore_axis_name="s",
            num_cores=_NC, num_subcores=_NS),
        scratch_types=[
            pltpu.VMEM((_NCH, _GCH), jnp.int32),
            pltpu.VMEM((_GCH, _DH), jnp.float32),
            pltpu.SemaphoreType.DMA,
        ],
    )


def kernel(x, mask, token_type_ids, key_optim, codebook, values):
    b, n, dim = x.shape
    h, k, dh = codebook.shape
    x2d = x.reshape(b * n, dim)
    xh = x.reshape(b * n, h, dh)
    x_sq = jnp.sum(xh * xh, axis=-1)
    cb_sq = jnp.sum(codebook * codebook, axis=-1)

    idx = _tc_argmin(
        x2d, codebook,
        cb_sq.reshape(h, 1, k),
        x_sq.T.reshape(h, b * n, 1),
    )

    idx2d = idx.reshape(_ROWS // _GCH, _GCH)
    rows = _sc_gather()(values.reshape(h * k, dh), idx2d)
    return rows.reshape(b, n, h * dh)
